# Initial kernel scaffold; baseline (speedup 1.0000x reference)
import jax
import jax.numpy as jnp
from jax.experimental import pallas as pl


def kernel(random_vector, cond, params, ei1, ei2, ei3, tbatch):
    raise NotImplementedError("write your pallas kernel here")


# speedup vs baseline: 94.6051x; 94.6051x over previous
import jax
import jax.numpy as jnp
from jax.experimental import pallas as pl

B = 256
D = 32
N_COND = 4
N_GLOBAL = 8
L2 = 16384
L3 = 131072
GRID = 16
P = L2 // GRID
PB_W = D + N_COND + N_GLOBAL

_K1_NAMES = []
for _il in range(2):
    for _p in ('hlv_W1', 'hlv_b1', 'hlv_W2', 'hlv_b2',
               'br_W1', 'br_b1', 'br_W2', 'br_b2',
               'cv_W1', 'cv_b1', 'cv_W2', 'cv_b2'):
        _K1_NAMES.append('%s_%d' % (_p, _il))
_K1_NAMES += ['hlv_W1_2', 'hlv_b1_2', 'hlv_W2_2', 'hlv_b2_2']
_K2_NAMES = ['br_W1_2', 'br_b1_2', 'br_W2_2', 'br_b2_2',
             'cv_W1_2', 'cv_b1_2', 'cv_W2_2', 'cv_b2_2']


def _lrelu(z):
    return jnp.where(z > 0, z, 0.01 * z)


def _ffn(x, w1, b1, w2, b2):
    h = _lrelu(jnp.dot(x, w1, preferred_element_type=jnp.float32) + b1)
    return jnp.dot(h, w2, preferred_element_type=jnp.float32) + b2


def _rep8(x):
    n, d = x.shape
    return jnp.broadcast_to(x[:, None, :], (n, 8, d)).reshape(n * 8, d)


def _branch(x, w1, b1, w2, b2):
    n = x.shape[0]
    h = _lrelu(jnp.dot(x, w1, preferred_element_type=jnp.float32) + b1)
    ys = [jnp.dot(h, w2[:, k * D:(k + 1) * D],
                  preferred_element_type=jnp.float32) + b2[:, k * D:(k + 1) * D]
          for k in range(8)]
    y = jnp.concatenate([yk[:, None, :] for yk in ys], axis=1)
    return y.reshape(n * 8, D)


def _k1_body(rv_ref, cond_ref, *rest):
    w = {n: rest[i][...] for i, n in enumerate(_K1_NAMES)}
    s_ref = rest[len(_K1_NAMES)]

    cond = cond_ref[...]
    x0 = jnp.concatenate([cond, rv_ref[:, N_COND:]], axis=1)
    g0 = _ffn(jnp.concatenate([x0, cond], 1),
              w['hlv_W1_0'], w['hlv_b1_0'], w['hlv_W2_0'], w['hlv_b2_0'])
    pb0 = jnp.concatenate([x0, cond, g0], 1)
    y1 = _branch(pb0, w['br_W1_0'], w['br_b1_0'], w['br_W2_0'], w['br_b2_0'])
    a0 = _ffn(pb0, w['cv_W1_0'], w['cv_b1_0'], w['cv_W2_0'], w['cv_b2_0'])
    u1 = _rep8(jnp.concatenate([x0, 2.0 * cond, 2.0 * g0], 1))
    u1 = jnp.concatenate([y1 + u1[:, :D], u1[:, D:]], 1)
    a1 = _ffn(u1, w['cv_W1_0'], w['cv_b1_0'], w['cv_W2_0'], w['cv_b2_0'])
    pooled1 = a1.reshape(B, 8, D).mean(axis=1)
    g1 = _ffn(jnp.concatenate([pooled1, cond], 1),
              w['hlv_W1_1'], w['hlv_b1_1'], w['hlv_W2_1'], w['hlv_b2_1'])
    cg1 = _rep8(jnp.concatenate([cond, g1], 1))
    pb1 = jnp.concatenate([a1, cg1], 1)
    y2 = _branch(pb1, w['br_W1_1'], w['br_b1_1'], w['br_W2_1'], w['br_b2_1'])
    b0 = _ffn(jnp.concatenate([a0, cond, g1], 1),
              w['cv_W1_1'], w['cv_b1_1'], w['cv_W2_1'], w['cv_b2_1'])
    a0r = _rep8(a0)
    u1b = jnp.concatenate([a1 + a0r, 2.0 * cg1], 1)
    b1 = _ffn(u1b, w['cv_W1_1'], w['cv_b1_1'], w['cv_W2_1'], w['cv_b2_1'])
    u2b = _rep8(jnp.concatenate([a1 + a0r, 3.0 * cg1], 1))
    u2b = jnp.concatenate([y2 + u2b[:, :D], u2b[:, D:]], 1)
    b2 = _ffn(u2b, w['cv_W1_1'], w['cv_b1_1'], w['cv_W2_1'], w['cv_b2_1'])
    pooled2 = b2.reshape(B, 64, D).mean(axis=1)
    g2 = _ffn(jnp.concatenate([pooled2, cond], 1),
              w['hlv_W1_2'], w['hlv_b1_2'], w['hlv_W2_2'], w['hlv_b2_2'])
    cg2 = _rep8(_rep8(jnp.concatenate([cond, g2], 1)))
    c2 = b2 + _rep8(b1 + _rep8(b0))
    s_ref[...] = jnp.concatenate([b2, cg2, c2, 4.0 * cg2], 1)


def _k2_body(s_ref, *rest):
    w = {n: rest[i][...] for i, n in enumerate(_K2_NAMES)}
    out_ref = rest[len(_K2_NAMES)]
    s = s_ref[...]
    y3 = _branch(s[:, :PB_W], w['br_W1_2'], w['br_b1_2'],
                 w['br_W2_2'], w['br_b2_2'])
    auxr = _rep8(s[:, PB_W:])
    u3 = jnp.concatenate([y3 + auxr[:, :D], auxr[:, D:]], 1)
    out_ref[...] = _ffn(u3, w['cv_W1_2'], w['cv_b1_2'],
                        w['cv_W2_2'], w['cv_b2_2'])


def kernel(random_vector, cond, params, ei1, ei2, ei3, tbatch):
    del ei1, ei2, ei3, tbatch
    get = lambda n: (params[n].reshape(1, -1)
                     if params[n].ndim == 1 else params[n])
    w1 = [get(n) for n in _K1_NAMES]
    w2 = [get(n) for n in _K2_NAMES]

    full = lambda a: pl.BlockSpec(a.shape, lambda *_: (0,) * a.ndim)
    s = pl.pallas_call(
        _k1_body,
        in_specs=[full(random_vector), full(cond)] + [full(x) for x in w1],
        out_specs=pl.BlockSpec((L2, 2 * PB_W), lambda *_: (0, 0)),
        out_shape=jax.ShapeDtypeStruct((L2, 2 * PB_W), jnp.float32),
    )(random_vector, cond, *w1)

    out = pl.pallas_call(
        _k2_body,
        grid=(GRID,),
        in_specs=[pl.BlockSpec((P, 2 * PB_W), lambda i: (i, 0))]
        + [full(x) for x in w2],
        out_specs=pl.BlockSpec((P * 8, D), lambda i: (i, 0)),
        out_shape=jax.ShapeDtypeStruct((L3, D), jnp.float32),
    )(s, *w2)
    return out


# speedup vs baseline: 110.9886x; 1.1732x over previous
import jax
import jax.numpy as jnp
from jax.experimental import pallas as pl

B = 256
D = 32
N_COND = 4
N_GLOBAL = 8
L2 = 16384
L3 = 131072
GRID = 16
P = L2 // GRID
PB_W = D + N_COND + N_GLOBAL
CH = 128

_K1_NAMES = []
for _il in range(2):
    for _p in ('hlv_W1', 'hlv_b1', 'hlv_W2', 'hlv_b2',
               'br_W1', 'br_b1', 'br_W2', 'br_b2',
               'cv_W1', 'cv_b1', 'cv_W2', 'cv_b2'):
        _K1_NAMES.append('%s_%d' % (_p, _il))
_K1_NAMES += ['hlv_W1_2', 'hlv_b1_2', 'hlv_W2_2', 'hlv_b2_2',
              'br_W2_2', 'br_b2_2', 'cv_W1_2', 'cv_b1_2', 'cv_W2_2']
_K2_NAMES = ['br_W1_2', 'br_b1_2', 'cv_W1_2', 'cv_b2_2']


def _lrelu(z):
    return jnp.where(z > 0, z, 0.01 * z)


def _dot(x, w):
    return jnp.dot(x, w, preferred_element_type=jnp.float32)


def _ffn(x, w1, b1, w2, b2):
    return _dot(_lrelu(_dot(x, w1) + b1), w2) + b2


def _rep8(x):
    n, d = x.shape
    return jnp.broadcast_to(x[:, None, :], (n, 8, d)).reshape(n * 8, d)


def _branch(x, w1, b1, w2, b2):
    n = x.shape[0]
    h = _lrelu(_dot(x, w1) + b1)
    ys = [_dot(h, w2[:, k * D:(k + 1) * D]) + b2[:, k * D:(k + 1) * D]
          for k in range(8)]
    y = jnp.concatenate([yk[:, None, :] for yk in ys], axis=1)
    return y.reshape(n * 8, D)


def _k1_body(rv_ref, cond_ref, *rest):
    w = {n: rest[i][...] for i, n in enumerate(_K1_NAMES)}
    s_ref, fw_ref, bd_ref = rest[len(_K1_NAMES):len(_K1_NAMES) + 3]

    cond = cond_ref[...]
    x0 = jnp.concatenate([cond, rv_ref[:, N_COND:]], axis=1)
    g0 = _ffn(jnp.concatenate([x0, cond], 1),
              w['hlv_W1_0'], w['hlv_b1_0'], w['hlv_W2_0'], w['hlv_b2_0'])
    pb0 = jnp.concatenate([x0, cond, g0], 1)
    y1 = _branch(pb0, w['br_W1_0'], w['br_b1_0'], w['br_W2_0'], w['br_b2_0'])
    a0 = _ffn(pb0, w['cv_W1_0'], w['cv_b1_0'], w['cv_W2_0'], w['cv_b2_0'])
    u1 = _rep8(jnp.concatenate([x0, 2.0 * cond, 2.0 * g0], 1))
    u1 = jnp.concatenate([y1 + u1[:, :D], u1[:, D:]], 1)
    a1 = _ffn(u1, w['cv_W1_0'], w['cv_b1_0'], w['cv_W2_0'], w['cv_b2_0'])
    pooled1 = a1.reshape(B, 8, D).mean(axis=1)
    g1 = _ffn(jnp.concatenate([pooled1, cond], 1),
              w['hlv_W1_1'], w['hlv_b1_1'], w['hlv_W2_1'], w['hlv_b2_1'])
    cg1 = _rep8(jnp.concatenate([cond, g1], 1))
    pb1 = jnp.concatenate([a1, cg1], 1)
    y2 = _branch(pb1, w['br_W1_1'], w['br_b1_1'], w['br_W2_1'], w['br_b2_1'])
    b0 = _ffn(jnp.concatenate([a0, cond, g1], 1),
              w['cv_W1_1'], w['cv_b1_1'], w['cv_W2_1'], w['cv_b2_1'])
    a0r = _rep8(a0)
    u1b = jnp.concatenate([a1 + a0r, 2.0 * cg1], 1)
    b1 = _ffn(u1b, w['cv_W1_1'], w['cv_b1_1'], w['cv_W2_1'], w['cv_b2_1'])
    u2b = _rep8(jnp.concatenate([a1 + a0r, 3.0 * cg1], 1))
    u2b = jnp.concatenate([y2 + u2b[:, :D], u2b[:, D:]], 1)
    b2 = _ffn(u2b, w['cv_W1_1'], w['cv_b1_1'], w['cv_W2_1'], w['cv_b2_1'])
    pooled2 = b2.reshape(B, 64, D).mean(axis=1)
    g2 = _ffn(jnp.concatenate([pooled2, cond], 1),
              w['hlv_W1_2'], w['hlv_b1_2'], w['hlv_W2_2'], w['hlv_b2_2'])
    cg2 = _rep8(_rep8(jnp.concatenate([cond, g2], 1)))
    c2 = b2 + _rep8(b1 + _rep8(b0))
    s_ref[...] = jnp.concatenate([b2, cg2, c2, 4.0 * cg2], 1)

    w1h = w['cv_W1_2'][:D, :]
    fs = [_dot(w['br_W2_2'][:, k * D:(k + 1) * D], w1h) for k in range(8)]
    cbs = [_dot(w['br_b2_2'][:, k * D:(k + 1) * D], w1h) + w['cv_b1_2']
           for k in range(8)]
    fw_ref[...] = jnp.concatenate(
        [jnp.concatenate(fs, 1), jnp.concatenate(cbs, 1)], 0)
    tiled = jnp.concatenate(
        [jnp.concatenate([w['cv_W2_2']] * 8, axis=0)] * 8, axis=1)
    r = jax.lax.broadcasted_iota(jnp.int32, (8 * CH, 8 * D), 0) // CH
    c = jax.lax.broadcasted_iota(jnp.int32, (8 * CH, 8 * D), 1) // D
    bd_ref[...] = jnp.where(r == c, tiled, 0.0)


def _k2_body(s_ref, fw_ref, bd_ref, *rest):
    w = {n: rest[i][...] for i, n in enumerate(_K2_NAMES)}
    out_ref = rest[len(_K2_NAMES)]
    s = s_ref[...]
    h = _lrelu(_dot(s[:, :PB_W], w['br_W1_2']) + w['br_b1_2'])
    t = _dot(s[:, PB_W:], w['cv_W1_2'])
    z = _dot(h, fw_ref[:64, :]) + jnp.concatenate([t] * 8, 1) + fw_ref[64:65, :]
    out_ref[...] = (_dot(_lrelu(z), bd_ref[...])
                    + jnp.concatenate([w['cv_b2_2']] * 8, 1))


def kernel(random_vector, cond, params, ei1, ei2, ei3, tbatch):
    del ei1, ei2, ei3, tbatch
    get = lambda n: (params[n].reshape(1, -1)
                     if params[n].ndim == 1 else params[n])
    w1 = [get(n) for n in _K1_NAMES]
    w2 = [get(n) for n in _K2_NAMES]

    full = lambda a: pl.BlockSpec(a.shape, lambda *_: (0,) * a.ndim)
    fullshape = lambda sh: pl.BlockSpec(sh, lambda *_: (0,) * len(sh))
    s, fw, bd = pl.pallas_call(
        _k1_body,
        in_specs=[full(random_vector), full(cond)] + [full(x) for x in w1],
        out_specs=[fullshape((L2, 2 * PB_W)), fullshape((65, 8 * CH)),
                   fullshape((8 * CH, 8 * D))],
        out_shape=[jax.ShapeDtypeStruct((L2, 2 * PB_W), jnp.float32),
                   jax.ShapeDtypeStruct((65, 8 * CH), jnp.float32),
                   jax.ShapeDtypeStruct((8 * CH, 8 * D), jnp.float32)],
    )(random_vector, cond, *w1)

    out = pl.pallas_call(
        _k2_body,
        grid=(GRID,),
        in_specs=[pl.BlockSpec((P, 2 * PB_W), lambda i: (i, 0)),
                  fullshape((65, 8 * CH)), fullshape((8 * CH, 8 * D))]
        + [full(x) for x in w2],
        out_specs=pl.BlockSpec((P, 8 * D), lambda i: (i, 0)),
        out_shape=jax.ShapeDtypeStruct((L2, 8 * D), jnp.float32),
    )(s, fw, bd, *w2)
    return out.reshape(L3, D)


# speedup vs baseline: 112.9681x; 1.0178x over previous
import jax
import jax.numpy as jnp
from jax.experimental import pallas as pl

B = 256
D = 32
N_COND = 4
N_GLOBAL = 8
L1 = 2048
L2 = 16384
L3 = 131072
GRID = 16
P = L2 // GRID
PB_W = D + N_COND + N_GLOBAL
CG = N_COND + N_GLOBAL
CH = 128

_K1_NAMES = []
for _il in range(2):
    for _p in ('hlv_W1', 'hlv_b1', 'hlv_W2', 'hlv_b2',
               'br_W1', 'br_b1', 'br_W2', 'br_b2',
               'cv_W1', 'cv_b1', 'cv_W2', 'cv_b2'):
        _K1_NAMES.append('%s_%d' % (_p, _il))
_K1_NAMES += ['hlv_W1_2', 'hlv_b1_2', 'hlv_W2_2', 'hlv_b2_2',
              'br_W2_2', 'br_b2_2', 'cv_W1_2', 'cv_b1_2', 'cv_W2_2']
_K2_NAMES = ['br_W1_2', 'br_b1_2', 'cv_W1_2', 'cv_b2_2']


def _lrelu(z):
    return jnp.where(z > 0, z, 0.01 * z)


def _dot(x, w):
    return jnp.dot(x, w, preferred_element_type=jnp.float32)


def _ffn(x, w1, b1, w2, b2):
    return _dot(_lrelu(_dot(x, w1) + b1), w2) + b2


def _rep8(x):
    n, d = x.shape
    return jnp.broadcast_to(x[:, None, :], (n, 8, d)).reshape(n * 8, d)


def _cat8(x):
    return jnp.concatenate([x] * 8, axis=1)


def _bdiag(w):
    a, b = w.shape
    t = jnp.concatenate([jnp.concatenate([w] * 8, axis=0)] * 8, axis=1)
    r = jax.lax.broadcasted_iota(jnp.int32, (8 * a, 8 * b), 0) // a
    c = jax.lax.broadcasted_iota(jnp.int32, (8 * a, 8 * b), 1) // b
    return jnp.where(r == c, t, 0.0)


def _branch(x, w1, b1, w2, b2):
    n = x.shape[0]
    h = _lrelu(_dot(x, w1) + b1)
    ys = [_dot(h, w2[:, k * D:(k + 1) * D]) + b2[:, k * D:(k + 1) * D]
          for k in range(8)]
    y = jnp.concatenate([yk[:, None, :] for yk in ys], axis=1)
    return y.reshape(n * 8, D)


def _k1_body(rv_ref, cond_ref, *rest):
    w = {n: rest[i][...] for i, n in enumerate(_K1_NAMES)}
    b2_ref, c2_ref, cg_ref, fw_ref, bd_ref = rest[len(_K1_NAMES):]

    cond = cond_ref[...]
    x0 = jnp.concatenate([cond, rv_ref[:, N_COND:]], axis=1)
    g0 = _ffn(jnp.concatenate([x0, cond], 1),
              w['hlv_W1_0'], w['hlv_b1_0'], w['hlv_W2_0'], w['hlv_b2_0'])
    pb0 = jnp.concatenate([x0, cond, g0], 1)
    y1 = _branch(pb0, w['br_W1_0'], w['br_b1_0'], w['br_W2_0'], w['br_b2_0'])
    a0 = _ffn(pb0, w['cv_W1_0'], w['cv_b1_0'], w['cv_W2_0'], w['cv_b2_0'])
    u1 = _rep8(jnp.concatenate([x0, 2.0 * cond, 2.0 * g0], 1))
    u1 = jnp.concatenate([y1 + u1[:, :D], u1[:, D:]], 1)
    a1 = _ffn(u1, w['cv_W1_0'], w['cv_b1_0'], w['cv_W2_0'], w['cv_b2_0'])
    pooled1 = a1.reshape(B, 8, D).mean(axis=1)
    g1 = _ffn(jnp.concatenate([pooled1, cond], 1),
              w['hlv_W1_1'], w['hlv_b1_1'], w['hlv_W2_1'], w['hlv_b2_1'])
    cg1 = _rep8(jnp.concatenate([cond, g1], 1))
    pb1 = jnp.concatenate([a1, cg1], 1)
    b0 = _ffn(jnp.concatenate([a0, cond, g1], 1),
              w['cv_W1_1'], w['cv_b1_1'], w['cv_W2_1'], w['cv_b2_1'])
    a0r = _rep8(a0)
    q = a1 + a0r
    u1b = jnp.concatenate([q, 2.0 * cg1], 1)
    b1 = _ffn(u1b, w['cv_W1_1'], w['cv_b1_1'], w['cv_W2_1'], w['cv_b2_1'])
    y2p = _ffn(pb1, w['br_W1_1'], w['br_b1_1'], w['br_W2_1'], w['br_b2_1'])
    u2p = y2p + _cat8(q)
    w1h = w['cv_W1_1'][:D, :]
    w1t = w['cv_W1_1'][D:, :]
    t2 = 3.0 * _dot(cg1, w1t) + w['cv_b1_1']
    hid = _lrelu(_dot(u2p, _bdiag(w1h)) + _cat8(t2))
    b2p = _dot(hid, _bdiag(w['cv_W2_1'])) + _cat8(w['cv_b2_1'])
    rs = b2p.reshape(B, 8, 8 * D).sum(axis=1)
    pooled2 = sum(rs[:, k * D:(k + 1) * D] for k in range(8)) * (1.0 / 64.0)
    g2 = _ffn(jnp.concatenate([pooled2, cond], 1),
              w['hlv_W1_2'], w['hlv_b1_2'], w['hlv_W2_2'], w['hlv_b2_2'])
    b2_ref[...] = b2p
    c2_ref[...] = b2p + _cat8(b1 + _rep8(b0))
    cg_ref[...] = jnp.concatenate([cond, g2], 1)

    w1h2 = w['cv_W1_2'][:D, :]
    fs = [_dot(w['br_W2_2'][:, k * D:(k + 1) * D], w1h2) for k in range(8)]
    cbs = [_dot(w['br_b2_2'][:, k * D:(k + 1) * D], w1h2) + w['cv_b1_2']
           for k in range(8)]
    fw_ref[...] = jnp.concatenate(
        [jnp.concatenate(fs, 1), jnp.concatenate(cbs, 1)], 0)
    bd_ref[...] = _bdiag(w['cv_W2_2'])


def _k2_body(b2_ref, c2_ref, cg_ref, fw_ref, bd_ref, *rest):
    w = {n: rest[i][...] for i, n in enumerate(_K2_NAMES)}
    out_ref = rest[len(_K2_NAMES)]
    cg = _rep8(_rep8(cg_ref[...]))
    pb = jnp.concatenate([b2_ref[...], cg], 1)
    h = _lrelu(_dot(pb, w['br_W1_2']) + w['br_b1_2'])
    w1h = w['cv_W1_2'][:D, :]
    w1t = w['cv_W1_2'][D:, :]
    t = _dot(c2_ref[...], w1h) + 4.0 * _dot(cg, w1t)
    z = _dot(h, fw_ref[:64, :]) + _cat8(t) + fw_ref[64:65, :]
    out_ref[...] = (_dot(_lrelu(z), bd_ref[...])
                    + _cat8(w['cv_b2_2']))


def kernel(random_vector, cond, params, ei1, ei2, ei3, tbatch):
    del ei1, ei2, ei3, tbatch
    get = lambda n: (params[n].reshape(1, -1)
                     if params[n].ndim == 1 else params[n])
    w1 = [get(n) for n in _K1_NAMES]
    w2 = [get(n) for n in _K2_NAMES]

    full = lambda a: pl.BlockSpec(a.shape, lambda *_: (0,) * a.ndim)
    fsh = lambda sh: pl.BlockSpec(sh, lambda *_: (0,) * len(sh))
    f32 = jnp.float32
    b2p, c2p, cg, fw, bd = pl.pallas_call(
        _k1_body,
        in_specs=[full(random_vector), full(cond)] + [full(x) for x in w1],
        out_specs=[fsh((L1, 8 * D)), fsh((L1, 8 * D)), fsh((B, CG)),
                   fsh((65, 8 * CH)), fsh((8 * CH, 8 * D))],
        out_shape=[jax.ShapeDtypeStruct((L1, 8 * D), f32),
                   jax.ShapeDtypeStruct((L1, 8 * D), f32),
                   jax.ShapeDtypeStruct((B, CG), f32),
                   jax.ShapeDtypeStruct((65, 8 * CH), f32),
                   jax.ShapeDtypeStruct((8 * CH, 8 * D), f32)],
    )(random_vector, cond, *w1)

    b2r = b2p.reshape(L2, D)
    c2r = c2p.reshape(L2, D)

    out = pl.pallas_call(
        _k2_body,
        grid=(GRID,),
        in_specs=[pl.BlockSpec((P, D), lambda i: (i, 0)),
                  pl.BlockSpec((P, D), lambda i: (i, 0)),
                  pl.BlockSpec((B // GRID, CG), lambda i: (i, 0)),
                  fsh((65, 8 * CH)), fsh((8 * CH, 8 * D))]
        + [full(x) for x in w2],
        out_specs=pl.BlockSpec((P, 8 * D), lambda i: (i, 0)),
        out_shape=jax.ShapeDtypeStruct((L2, 8 * D), f32),
    )(b2r, c2r, cg, fw, bd, *w2)
    return out.reshape(L3, D)
